# Initial kernel scaffold; baseline (speedup 1.0000x reference)
import functools

import jax
import jax.numpy as jnp
from jax import lax
from jax.experimental import pallas as pl
from jax.experimental.pallas import tpu as pltpu
from jax.experimental.pallas import tpu_sc as plsc

N = 10000
D = 256
DH = 128
E = 160000
EMB = 500

NS = 16
CH = 128
EP_TILE = 10240
E_PAD = EP_TILE * NS
NCHUNK = EP_TILE // CH
RPT = N // NS
RCH = 125


def _sc_body(x_lo, x_hi, src, dst, w, out,
             agg_sh, idx_s, idx_d, wbuf, rows, rbuf, sem):
    c = lax.axis_index("c")
    s = lax.axis_index("s")

    def zrow(r, carry):
        for j in range(DH // 16):
            rbuf[r, pl.ds(j * 16, 16)] = jnp.zeros((16,), jnp.float32)
        return carry
    lax.fori_loop(0, RCH, zrow, 0)
    for k in range(RPT // RCH):
        pltpu.sync_copy(rbuf, agg_sh.at[pl.ds(s * RPT + k * RCH, RCH)])
    plsc.subcore_barrier()

    def process(xc):
        def chunk(g, carry):
            base = s * EP_TILE + g * CH
            pltpu.sync_copy(src.at[pl.ds(base, CH)], idx_s)
            pltpu.sync_copy(dst.at[pl.ds(base, CH)], idx_d)
            pltpu.sync_copy(w.at[pl.ds(base, CH)], wbuf)
            pltpu.async_copy(xc.at[idx_s], rows, sem).wait()
            def edge(e, cy):
                wv = jnp.full((16,), wbuf[e], jnp.float32)
                for j in range(DH // 16):
                    sl = pl.ds(j * 16, 16)
                    rows[e, sl] = rows[e, sl] * wv
                return cy
            lax.fori_loop(0, CH, edge, 0)
            pltpu.sync_copy(rows, agg_sh.at[idx_d], add=True)
            return carry
        lax.fori_loop(0, NCHUNK, chunk, 0)

    @pl.when(c == 0)
    def _():
        process(x_lo)

    @pl.when(c == 1)
    def _():
        process(x_hi)

    plsc.subcore_barrier()
    for k in range(RPT // RCH):
        r0 = s * RPT + k * RCH
        pltpu.sync_copy(agg_sh.at[pl.ds(r0, RCH)], rbuf)
        pltpu.sync_copy(rbuf, out.at[c, pl.ds(r0, RCH)])


_sc_call = pl.kernel(
    _sc_body,
    out_type=jax.ShapeDtypeStruct((2, N, DH), jnp.float32),
    mesh=plsc.VectorSubcoreMesh(core_axis_name="c", subcore_axis_name="s"),
    scratch_types=[
        pltpu.VMEM_SHARED((N, DH), jnp.float32),
        pltpu.VMEM((CH,), jnp.int32),
        pltpu.VMEM((CH,), jnp.int32),
        pltpu.VMEM((CH,), jnp.float32),
        pltpu.VMEM((CH, DH), jnp.float32),
        pltpu.VMEM((RCH, DH), jnp.float32),
        pltpu.SemaphoreType.DMA,
    ],
)

RB = 500


def _tc_body(agg_ref, we_ref, be_ref, wd_ref, bd_ref, pred_ref, reg_ref):
    a_lo = agg_ref[0]
    a_hi = agg_ref[1]
    z = (jnp.dot(a_lo, we_ref[:DH, :], preferred_element_type=jnp.float32)
         + jnp.dot(a_hi, we_ref[DH:, :], preferred_element_type=jnp.float32)
         + be_ref[0][None, :])
    h = jax.nn.sigmoid(z)
    p = (jnp.dot(h, wd_ref[...], preferred_element_type=jnp.float32)
         + bd_ref[0][None, :])
    pred_ref[...] = p

    @pl.when(pl.program_id(0) == 0)
    def _():
        reg_ref[0, 0] = (jnp.sum(we_ref[...] * we_ref[...])
                         + jnp.sum(wd_ref[...] * wd_ref[...]))


_tc_call = pl.pallas_call(
    _tc_body,
    grid=(N // RB,),
    in_specs=[
        pl.BlockSpec((2, RB, DH), lambda i: (0, i, 0)),
        pl.BlockSpec((D, EMB), lambda i: (0, 0)),
        pl.BlockSpec((1, EMB), lambda i: (0, 0)),
        pl.BlockSpec((EMB, D), lambda i: (0, 0)),
        pl.BlockSpec((1, D), lambda i: (0, 0)),
    ],
    out_specs=[
        pl.BlockSpec((RB, D), lambda i: (i, 0)),
        pl.BlockSpec((1, 1), lambda i: (0, 0)),
    ],
    out_shape=[
        jax.ShapeDtypeStruct((N, D), jnp.float32),
        jax.ShapeDtypeStruct((1, 1), jnp.float32),
    ],
)


@jax.jit
def kernel(x, edge_index, edge_weight, W_enc, b_enc, W_dec, b_dec):
    src = edge_index[0].astype(jnp.int32)
    dst = edge_index[1].astype(jnp.int32)
    pad = E_PAD - E
    src_p = jnp.concatenate([src, jnp.zeros((pad,), jnp.int32)])
    dst_p = jnp.concatenate([dst, jnp.zeros((pad,), jnp.int32)])
    w_p = jnp.concatenate([edge_weight.astype(jnp.float32),
                           jnp.zeros((pad,), jnp.float32)])
    x_lo = x[:, :DH]
    x_hi = x[:, DH:]
    agg_halves = _sc_call(x_lo, x_hi, src_p, dst_p, w_p)
    pred, reg = _tc_call(agg_halves, W_enc, b_enc.reshape(1, EMB),
                         W_dec, b_dec.reshape(1, D))
    return (x, pred, reg[0, 0])


# speedup vs baseline: 3.3293x; 3.3293x over previous
import jax
import jax.numpy as jnp
from jax import lax
from jax.experimental import pallas as pl
from jax.experimental.pallas import tpu as pltpu
from jax.experimental.pallas import tpu_sc as plsc

N = 10000
D = 256
DH = 128
E = 160000
EMB = 500

NS = 16
CH = 128
CHH = CH // 2
EP_TILE = 10240
E_PAD = EP_TILE * NS
NCHUNK = EP_TILE // CH
RCH = 80
NDCH = N // RCH


def _sc_body(x_lo, x_hi, packed, out,
             agg_sh, mbuf0, mbuf1, mbuf2, mbuf3, rows0, rows1,
             gsem0a, gsem0b, gsem1a, gsem1b, ssem0, ssem1,
             msem0, msem1, msem2, msem3):
    c = lax.axis_index("c")
    s = lax.axis_index("s")

    nt = (NDCH - s + NS - 1) // NS

    def zrow(r, carry):
        for j in range(DH // 16):
            rows0[r, pl.ds(j * 16, 16)] = jnp.zeros((16,), jnp.float32)
        return carry
    lax.fori_loop(0, RCH, zrow, 0)

    def zchunk(t, carry):
        cid = s + t * NS
        pltpu.sync_copy(rows0.at[pl.ds(0, RCH)],
                        agg_sh.at[pl.ds(cid * RCH, RCH)])
        return carry
    lax.fori_loop(0, nt, zchunk, 0)
    plsc.subcore_barrier()

    def process(xc):
        gbufs = ((rows0, gsem0a, gsem0b, ssem0), (rows1, gsem1a, gsem1b,
                                                  ssem1))
        mbufs = ((mbuf0, msem0), (mbuf1, msem1), (mbuf2, msem2),
                 (mbuf3, msem3))

        def meta_start(g, mb, ms):
            pltpu.async_copy(packed.at[s * NCHUNK + g], mb, ms)

        def meta_wait(mb, ms):
            pltpu.make_async_copy(packed.at[0], mb, ms).wait()

        def gather_start(mb, rows, sa, sb):
            pltpu.async_copy(xc.at[mb.at[0, pl.ds(0, CHH)]],
                             rows.at[pl.ds(0, CHH)], sa)
            pltpu.async_copy(xc.at[mb.at[0, pl.ds(CHH, CHH)]],
                             rows.at[pl.ds(CHH, CHH)], sb)

        def gather_wait(rows, sa, sb):
            pltpu.make_async_copy(xc.at[mbuf0.at[0, pl.ds(0, CHH)]],
                                  rows.at[pl.ds(0, CHH)], sa).wait()
            pltpu.make_async_copy(xc.at[mbuf0.at[0, pl.ds(0, CHH)]],
                                  rows.at[pl.ds(CHH, CHH)], sb).wait()

        def mul_rows(rows, mb):
            def edge16(k, cy):
                w16 = lax.bitcast_convert_type(mb[2, pl.ds(k * 16, 16)],
                                               jnp.float32)
                for l in range(16):
                    e = k * 16 + l
                    wv = jnp.full((16,), w16[l], jnp.float32)
                    for j in range(DH // 16):
                        sl = pl.ds(j * 16, 16)
                        rows[e, sl] = rows[e, sl] * wv
                return cy
            lax.fori_loop(0, CH // 16, edge16, 0)

        meta_start(0, mbuf0, msem0)
        meta_start(1, mbuf1, msem1)
        meta_wait(mbuf0, msem0)
        gather_start(mbuf0, rows0, gsem0a, gsem0b)

        def step(t, carry):
            for p in range(4):
                g = 4 * t + p
                rows, ga, gb, ssem = gbufs[p % 2]
                rows_o, ga_o, gb_o, ssem_o = gbufs[1 - p % 2]
                mb, ms = mbufs[p]
                mb1, ms1 = mbufs[(p + 1) % 4]
                mb2, ms2 = mbufs[(p + 2) % 4]
                gather_wait(rows, ga, gb)
                @pl.when(g >= 1)
                def _():
                    pltpu.make_async_copy(
                        rows_o, agg_sh.at[mbuf0.at[1]], ssem_o).wait()

                @pl.when(g + 1 < NCHUNK)
                def _():
                    meta_wait(mb1, ms1)
                    gather_start(mb1, rows_o, ga_o, gb_o)

                @pl.when(g + 2 < NCHUNK)
                def _():
                    meta_start(g + 2, mb2, ms2)
                mul_rows(rows, mb)
                pltpu.async_copy(rows, agg_sh.at[mb.at[1]], ssem, add=True)
            return carry
        lax.fori_loop(0, NCHUNK // 4, step, 0)
        pltpu.make_async_copy(rows1, agg_sh.at[mbuf0.at[1]], ssem1).wait()

    @pl.when(c == 0)
    def _():
        process(x_lo)

    @pl.when(c == 1)
    def _():
        process(x_hi)

    plsc.subcore_barrier()

    def dchunk(t, carry):
        r0 = (s + t * NS) * RCH
        pltpu.sync_copy(agg_sh.at[pl.ds(r0, RCH)], out.at[c, pl.ds(r0, RCH)])
        return carry
    lax.fori_loop(0, nt, dchunk, 0)


_sc_call = pl.kernel(
    _sc_body,
    out_type=jax.ShapeDtypeStruct((2, N, DH), jnp.float32),
    mesh=plsc.VectorSubcoreMesh(core_axis_name="c", subcore_axis_name="s"),
    scratch_types=(
        [pltpu.VMEM_SHARED((N, DH), jnp.float32)]
        + [pltpu.VMEM((3, CH), jnp.int32)] * 4
        + [pltpu.VMEM((CH, DH), jnp.float32)] * 2
        + [pltpu.SemaphoreType.DMA] * 10
    ),
)

RB = 1000


def _tc_body(agg_ref, we_ref, be_ref, wd_ref, bd_ref, pred_ref, reg_ref):
    a_lo = agg_ref[0]
    a_hi = agg_ref[1]
    z = (jnp.dot(a_lo, we_ref[:DH, :], preferred_element_type=jnp.float32)
         + jnp.dot(a_hi, we_ref[DH:, :], preferred_element_type=jnp.float32)
         + be_ref[0][None, :])
    h = jax.nn.sigmoid(z)
    p = (jnp.dot(h, wd_ref[...], preferred_element_type=jnp.float32)
         + bd_ref[0][None, :])
    pred_ref[...] = p

    @pl.when(pl.program_id(0) == 0)
    def _():
        reg = (jnp.sum(we_ref[...] * we_ref[...])
               + jnp.sum(wd_ref[...] * wd_ref[...]))
        reg_ref[...] = reg[None, None]


_tc_call = pl.pallas_call(
    _tc_body,
    grid=(N // RB,),
    in_specs=[
        pl.BlockSpec((2, RB, DH), lambda i: (0, i, 0)),
        pl.BlockSpec((D, EMB), lambda i: (0, 0)),
        pl.BlockSpec((1, EMB), lambda i: (0, 0)),
        pl.BlockSpec((EMB, D), lambda i: (0, 0)),
        pl.BlockSpec((1, D), lambda i: (0, 0)),
    ],
    out_specs=[
        pl.BlockSpec((RB, D), lambda i: (i, 0)),
        pl.BlockSpec((1, 1), lambda i: (0, 0)),
    ],
    out_shape=[
        jax.ShapeDtypeStruct((N, D), jnp.float32),
        jax.ShapeDtypeStruct((1, 1), jnp.float32),
    ],
)


@jax.jit
def kernel(x, edge_index, edge_weight, W_enc, b_enc, W_dec, b_dec):
    src = edge_index[0].astype(jnp.int32)
    dst = edge_index[1].astype(jnp.int32)
    pad = E_PAD - E
    src_p = jnp.concatenate([src, jnp.zeros((pad,), jnp.int32)])
    dst_p = jnp.concatenate([dst, jnp.zeros((pad,), jnp.int32)])
    w_p = jnp.concatenate([edge_weight.astype(jnp.float32),
                           jnp.zeros((pad,), jnp.float32)])
    wbits = lax.bitcast_convert_type(w_p, jnp.int32)
    packed = jnp.stack([src_p.reshape(-1, CH), dst_p.reshape(-1, CH),
                        wbits.reshape(-1, CH)], axis=1)
    x_lo = x[:, :DH]
    x_hi = x[:, DH:]
    agg_halves = _sc_call(x_lo, x_hi, packed)
    pred, reg = _tc_call(agg_halves, W_enc, b_enc.reshape(1, EMB),
                         W_dec, b_dec.reshape(1, D))
    return (x, pred, reg[0, 0])


# speedup vs baseline: 3.4068x; 1.0233x over previous
import jax
import jax.numpy as jnp
from jax import lax
from jax.experimental import pallas as pl
from jax.experimental.pallas import tpu as pltpu
from jax.experimental.pallas import tpu_sc as plsc

N = 10000
D = 256
DH = 128
E = 160000
EMB = 500

NS = 16
CH = 128
CHH = CH // 2
EP_TILE = 10240
E_PAD = EP_TILE * NS
NCHUNK = EP_TILE // CH
RCH = 80
NDCH = N // RCH


def _sc_body(x2, packed, out,
             agg_sh, mbuf0, mbuf1, mbuf2, mbuf3, rows0, rows1,
             gsem0a, gsem0b, gsem1a, gsem1b, ssem0, ssem1,
             msem0, msem1, msem2, msem3):
    c = lax.axis_index("c")
    s = lax.axis_index("s")

    nt = (NDCH - s + NS - 1) // NS

    def process(xc):
        gbufs = ((rows0, gsem0a, gsem0b, ssem0), (rows1, gsem1a, gsem1b,
                                                  ssem1))
        mbufs = ((mbuf0, msem0), (mbuf1, msem1), (mbuf2, msem2),
                 (mbuf3, msem3))

        def meta_start(g, mb, ms):
            pltpu.async_copy(packed.at[s * NCHUNK + g], mb, ms)

        def meta_wait(mb, ms):
            pltpu.make_async_copy(packed.at[0], mb, ms).wait()

        def idx_fix(mb):
            for kk in range(CH // 16):
                sl = pl.ds(kk * 16, 16)
                mb[0, sl] = mb[0, sl] * 2 + c

        def gather_start(mb, rows, sa, sb):
            pltpu.async_copy(xc.at[mb.at[0, pl.ds(0, CHH)]],
                             rows.at[pl.ds(0, CHH)], sa)
            pltpu.async_copy(xc.at[mb.at[0, pl.ds(CHH, CHH)]],
                             rows.at[pl.ds(CHH, CHH)], sb)

        def gather_wait(rows, sa, sb):
            pltpu.make_async_copy(xc.at[mbuf0.at[0, pl.ds(0, CHH)]],
                                  rows.at[pl.ds(0, CHH)], sa).wait()
            pltpu.make_async_copy(xc.at[mbuf0.at[0, pl.ds(0, CHH)]],
                                  rows.at[pl.ds(CHH, CHH)], sb).wait()

        def mul_rows(rows, mb):
            def edge16(k, cy):
                w16 = lax.bitcast_convert_type(mb[2, pl.ds(k * 16, 16)],
                                               jnp.float32)
                for l in range(16):
                    e = k * 16 + l
                    wv = jnp.full((16,), w16[l], jnp.float32)
                    for j in range(DH // 16):
                        sl = pl.ds(j * 16, 16)
                        rows[e, sl] = rows[e, sl] * wv
                return cy
            lax.fori_loop(0, CH // 16, edge16, 0)

        meta_start(0, mbuf0, msem0)
        meta_start(1, mbuf1, msem1)
        meta_wait(mbuf0, msem0)
        idx_fix(mbuf0)
        gather_start(mbuf0, rows0, gsem0a, gsem0b)

        def zrow(r, carry):
            for j in range(DH // 16):
                rows1[r, pl.ds(j * 16, 16)] = jnp.zeros((16,), jnp.float32)
            return carry
        lax.fori_loop(0, RCH, zrow, 0)

        def zchunk(t, carry):
            cid = s + t * NS
            pltpu.sync_copy(rows1.at[pl.ds(0, RCH)],
                            agg_sh.at[pl.ds(cid * RCH, RCH)])
            return carry
        lax.fori_loop(0, nt, zchunk, 0)
        plsc.subcore_barrier()

        def step(t, carry):
            for p in range(4):
                g = 4 * t + p
                rows, ga, gb, ssem = gbufs[p % 2]
                rows_o, ga_o, gb_o, ssem_o = gbufs[1 - p % 2]
                mb, ms = mbufs[p]
                mb1, ms1 = mbufs[(p + 1) % 4]
                mb2, ms2 = mbufs[(p + 2) % 4]
                gather_wait(rows, ga, gb)
                @pl.when(g >= 1)
                def _():
                    pltpu.make_async_copy(
                        rows_o, agg_sh.at[mbuf0.at[1]], ssem_o).wait()

                @pl.when(g + 1 < NCHUNK)
                def _():
                    meta_wait(mb1, ms1)
                    idx_fix(mb1)
                    gather_start(mb1, rows_o, ga_o, gb_o)

                @pl.when(g + 2 < NCHUNK)
                def _():
                    meta_start(g + 2, mb2, ms2)
                mul_rows(rows, mb)
                pltpu.async_copy(rows, agg_sh.at[mb.at[1]], ssem, add=True)
            return carry
        lax.fori_loop(0, NCHUNK // 4, step, 0)
        pltpu.make_async_copy(rows1, agg_sh.at[mbuf0.at[1]], ssem1).wait()

    process(x2)

    plsc.subcore_barrier()

    def dchunk(t, carry):
        r0 = (s + t * NS) * RCH
        pltpu.sync_copy(agg_sh.at[pl.ds(r0, RCH)], out.at[c, pl.ds(r0, RCH)])
        return carry
    lax.fori_loop(0, nt, dchunk, 0)


_sc_call = pl.kernel(
    _sc_body,
    out_type=jax.ShapeDtypeStruct((2, N, DH), jnp.float32),
    mesh=plsc.VectorSubcoreMesh(core_axis_name="c", subcore_axis_name="s"),
    scratch_types=(
        [pltpu.VMEM_SHARED((N, DH), jnp.float32)]
        + [pltpu.VMEM((3, CH), jnp.int32)] * 4
        + [pltpu.VMEM((CH, DH), jnp.float32)] * 2
        + [pltpu.SemaphoreType.DMA] * 10
    ),
)

RB = 1000


def _tc_body(agg_ref, we_ref, be_ref, wd_ref, bd_ref, pred_ref, reg_ref):
    a_lo = agg_ref[0]
    a_hi = agg_ref[1]
    z = (jnp.dot(a_lo, we_ref[:DH, :], preferred_element_type=jnp.float32)
         + jnp.dot(a_hi, we_ref[DH:, :], preferred_element_type=jnp.float32)
         + be_ref[0][None, :])
    h = jax.nn.sigmoid(z)
    p = (jnp.dot(h, wd_ref[...], preferred_element_type=jnp.float32)
         + bd_ref[0][None, :])
    pred_ref[...] = p

    @pl.when(pl.program_id(0) == 0)
    def _():
        reg = (jnp.sum(we_ref[...] * we_ref[...])
               + jnp.sum(wd_ref[...] * wd_ref[...]))
        reg_ref[...] = reg[None, None]


_tc_call = pl.pallas_call(
    _tc_body,
    grid=(N // RB,),
    in_specs=[
        pl.BlockSpec((2, RB, DH), lambda i: (0, i, 0)),
        pl.BlockSpec((D, EMB), lambda i: (0, 0)),
        pl.BlockSpec((1, EMB), lambda i: (0, 0)),
        pl.BlockSpec((EMB, D), lambda i: (0, 0)),
        pl.BlockSpec((1, D), lambda i: (0, 0)),
    ],
    out_specs=[
        pl.BlockSpec((RB, D), lambda i: (i, 0)),
        pl.BlockSpec((1, 1), lambda i: (0, 0)),
    ],
    out_shape=[
        jax.ShapeDtypeStruct((N, D), jnp.float32),
        jax.ShapeDtypeStruct((1, 1), jnp.float32),
    ],
)


@jax.jit
def kernel(x, edge_index, edge_weight, W_enc, b_enc, W_dec, b_dec):
    src = edge_index[0].astype(jnp.int32)
    dst = edge_index[1].astype(jnp.int32)
    pad = E_PAD - E
    src_p = jnp.concatenate([src, jnp.zeros((pad,), jnp.int32)])
    dst_p = jnp.concatenate([dst, jnp.zeros((pad,), jnp.int32)])
    w_p = jnp.concatenate([edge_weight.astype(jnp.float32),
                           jnp.zeros((pad,), jnp.float32)])
    wbits = lax.bitcast_convert_type(w_p, jnp.int32)
    packed = jnp.stack([src_p.reshape(-1, CH), dst_p.reshape(-1, CH),
                        wbits.reshape(-1, CH)], axis=1)
    x2 = x.reshape(2 * N, DH)
    agg_halves = _sc_call(x2, packed)
    pred, reg = _tc_call(agg_halves, W_enc, b_enc.reshape(1, EMB),
                         W_dec, b_dec.reshape(1, D))
    return (x, pred, reg[0, 0])
